# baseline (speedup 1.0000x reference)
import functools

import jax
import jax.numpy as jnp
from jax import lax
from jax.experimental import pallas as pl
from jax.experimental.pallas import tpu as pltpu
from jax.experimental.pallas import tpu_sc as plsc

ALPHA = 0.5
BETA = 0.5
EPS = 1e-06
C = 19
B = 8
HW = 512 * 512
P = B * HW

WIN_ROWS = 8
WIN = WIN_ROWS * 512
NWIN = HW // WIN
NBUF = 4
CT = 15


def _sc_partials(nw):
    ncg = nw // B
    nslab = -(-(C - CT) // ncg)

    mesh = plsc.VectorSubcoreMesh(core_axis_name="c", subcore_axis_name="s")

    @functools.partial(
        pl.kernel,
        out_type=[
            jax.ShapeDtypeStruct((nw, 16), jnp.float32),
            jax.ShapeDtypeStruct((nw, 16), jnp.float32),
        ],
        mesh=mesh,
        compiler_params=pltpu.CompilerParams(needs_layout_passes=False),
        scratch_types=(
            [pltpu.VMEM((WIN_ROWS, 512), jnp.int32) for _ in range(NBUF)]
            + [pltpu.VMEM(((nslab + 1) * WIN_ROWS, 512), jnp.float32)
               for _ in range(NBUF)]
            + [pltpu.VMEM((16,), jnp.float32), pltpu.VMEM((16,), jnp.float32)]
            + [pltpu.SemaphoreType.DMA for _ in range(NBUF)]
        ),
    )
    def body(yp_ref, yt_ref, outg_ref, outs_ref, *refs):
        labs = refs[:NBUF]
        preds = refs[NBUF:2 * NBUF]
        stg_g, stg_s = refs[2 * NBUF], refs[2 * NBUF + 1]
        sems = refs[2 * NBUF + 2:]

        nc = jax.lax.axis_size("c")
        wid = lax.axis_index("s") * nc + lax.axis_index("c")
        b = wid // ncg
        cg = wid % ncg
        c_lo = CT + cg * nslab
        lab_row0 = b * 512

        iota = lax.iota(jnp.int32, 16)

        def start(w, p):
            rb = w * WIN_ROWS
            pltpu.async_copy(
                yt_ref.at[pl.ds(lab_row0 + rb, WIN_ROWS)], labs[p], sems[p])
            for j in range(nslab):
                c_src = jnp.minimum(c_lo + j, C - 1)
                row = (b * C + c_src) * 512 + rb
                pltpu.async_copy(
                    yp_ref.at[pl.ds(row, WIN_ROWS)],
                    preds[p].at[pl.ds(j * WIN_ROWS, WIN_ROWS)], sems[p])

        def wait(p):
            pltpu.make_async_copy(
                yp_ref.at[pl.ds(0, nslab * WIN_ROWS)],
                preds[p].at[pl.ds(0, nslab * WIN_ROWS)], sems[p]).wait()
            pltpu.make_async_copy(
                yt_ref.at[pl.ds(0, WIN_ROWS)], labs[p], sems[p]).wait()

        def compute(p, car):
            lab_ref, pred_ref = labs[p], preds[p]

            def row_body(r, c5):
                s_acc = c5[0]
                g = list(c5[1:])
                for l in range(32):
                    lab16 = lab_ref[r, pl.ds(l * 16, 16)]
                    jrel = plsc.bitcast(lab16 - c_lo, jnp.uint32)
                    jsel = plsc.bitcast(
                        jnp.minimum(jrel, jnp.uint32(nslab)), jnp.int32)
                    row16 = (jsel << 3) + r
                    col16 = iota + (l * 16)
                    v16 = plsc.load_gather(pred_ref, [row16, col16])
                    g[l % 4] = g[l % 4] + v16
                    s_acc = s_acc + lab16
                return (s_acc, g[0], g[1], g[2], g[3])

            return lax.fori_loop(0, WIN_ROWS, row_body, car)

        def zero_junk(p):
            zf = jnp.zeros((16,), jnp.float32)

            def zrow(r, _):
                for l in range(32):
                    preds[p][nslab * WIN_ROWS + r, pl.ds(l * 16, 16)] = zf
                return 0

            lax.fori_loop(0, WIN_ROWS, zrow, 0)

        for p in range(NBUF):
            zero_junk(p)
            start(p, p)

        zf = jnp.zeros((16,), jnp.float32)
        car = (jnp.zeros((16,), jnp.int32), zf, zf, zf, zf)

        def outer(w4, car):
            for p in range(NBUF):
                w = w4 * NBUF + p
                wait(p)
                car = compute(p, car)

                @pl.when(w + NBUF < NWIN)
                def _():
                    start(w + NBUF, p)
            return car

        car = lax.fori_loop(0, NWIN // NBUF, outer, car)

        s_acc, g0, g1, g2, g3 = car
        g_tot = (g0 + g1) + (g2 + g3)
        cg_v = jnp.full((16,), cg, jnp.int32)
        s_fin = jnp.where(cg_v == 0, s_acc, 0).astype(jnp.float32)
        stg_g[...] = g_tot
        stg_s[...] = s_fin
        pltpu.sync_copy(stg_g, outg_ref.at[wid])
        pltpu.sync_copy(stg_s, outs_ref.at[wid])

    return body


def _tc_dense_kernel(yp_ref, yt_ref, g_ref):
    b = pl.program_id(0)
    c = pl.program_id(1)

    @pl.when((b == 0) & (c == 0))
    def _():
        g_ref[0, 0] = 0.0

    yp = yp_ref[0, 0]
    yt = yt_ref[0, 0]
    g_ref[0, 0] += jnp.sum(jnp.where(yt == c, yp, 0.0))


def _combine_kernel(g_ref, s_ref, gtc_ref, o_ref):
    g = jnp.sum(g_ref[...]) + gtc_ref[0, 0]
    s = jnp.sum(s_ref[...])
    denom = 10.0 * s + BETA * (float(P) - g) + EPS
    o_ref[0, 0] = 1.0 - s / denom


def kernel(y_pred, y_true):
    info = plsc.get_sparse_core_info()
    nw = info.num_cores * info.num_subcores

    yp2 = y_pred.reshape(B * C * 512, 512)
    yt2 = y_true.reshape(B * 512, 512)

    gpart, spart = _sc_partials(nw)(yp2, yt2)

    gtc = pl.pallas_call(
        _tc_dense_kernel,
        grid=(B, CT),
        in_specs=[
            pl.BlockSpec((1, 1, 512, 512), lambda b, c: (b, c, 0, 0)),
            pl.BlockSpec((1, 1, 512, 512), lambda b, c: (b, 0, 0, 0)),
        ],
        out_specs=pl.BlockSpec((1, 1), lambda b, c: (0, 0),
                               memory_space=pltpu.SMEM),
        out_shape=jax.ShapeDtypeStruct((1, 1), jnp.float32),
    )(y_pred, y_true)

    out = pl.pallas_call(
        _combine_kernel,
        in_specs=[
            pl.BlockSpec((nw, 16), lambda: (0, 0)),
            pl.BlockSpec((nw, 16), lambda: (0, 0)),
            pl.BlockSpec(memory_space=pltpu.SMEM),
        ],
        out_shape=jax.ShapeDtypeStruct((1, 1), jnp.float32),
        out_specs=pl.BlockSpec(memory_space=pltpu.SMEM),
    )(gpart, spart, gtc)
    return out.reshape(())


# speedup vs baseline: 1.6239x; 1.6239x over previous
import functools

import jax
import jax.numpy as jnp
from jax import lax
from jax.experimental import pallas as pl
from jax.experimental.pallas import tpu as pltpu
from jax.experimental.pallas import tpu_sc as plsc

ALPHA = 0.5
BETA = 0.5
EPS = 1e-06
C = 19
B = 8
HW = 512 * 512
P = B * HW

WIN_ROWS = 8
WIN = WIN_ROWS * 512
NWIN = HW // WIN
NBUF = 4
CT = 15


def _sc_partials(nw):
    ncg = nw // B
    nslab = -(-(C - CT) // ncg)

    mesh = plsc.VectorSubcoreMesh(core_axis_name="c", subcore_axis_name="s")

    @functools.partial(
        pl.kernel,
        out_type=[
            jax.ShapeDtypeStruct((nw, 16), jnp.float32),
            jax.ShapeDtypeStruct((nw, 16), jnp.float32),
        ],
        mesh=mesh,
        compiler_params=pltpu.CompilerParams(needs_layout_passes=False),
        scratch_types=(
            [pltpu.VMEM((WIN_ROWS, 512), jnp.int32) for _ in range(NBUF)]
            + [pltpu.VMEM(((nslab + 1) * WIN_ROWS, 512), jnp.float32)
               for _ in range(NBUF)]
            + [pltpu.VMEM((16,), jnp.float32), pltpu.VMEM((16,), jnp.float32)]
            + [pltpu.SemaphoreType.DMA for _ in range(NBUF)]
        ),
    )
    def body(yp_ref, yt_ref, outg_ref, outs_ref, *refs):
        labs = refs[:NBUF]
        preds = refs[NBUF:2 * NBUF]
        stg_g, stg_s = refs[2 * NBUF], refs[2 * NBUF + 1]
        sems = refs[2 * NBUF + 2:]

        nc = jax.lax.axis_size("c")
        wid = lax.axis_index("s") * nc + lax.axis_index("c")
        b = wid // ncg
        cg = wid % ncg
        c_lo = CT + cg * nslab
        lab_row0 = b * 512

        iota = lax.iota(jnp.int32, 16)

        def start(w, p):
            rb = w * WIN_ROWS
            pltpu.async_copy(
                yt_ref.at[pl.ds(lab_row0 + rb, WIN_ROWS)], labs[p], sems[p])
            for j in range(nslab):
                c_src = jnp.minimum(c_lo + j, C - 1)
                row = (b * C + c_src) * 512 + rb
                pltpu.async_copy(
                    yp_ref.at[pl.ds(row, WIN_ROWS)],
                    preds[p].at[pl.ds(j * WIN_ROWS, WIN_ROWS)], sems[p])

        def wait(p):
            pltpu.make_async_copy(
                yp_ref.at[pl.ds(0, nslab * WIN_ROWS)],
                preds[p].at[pl.ds(0, nslab * WIN_ROWS)], sems[p]).wait()
            pltpu.make_async_copy(
                yt_ref.at[pl.ds(0, WIN_ROWS)], labs[p], sems[p]).wait()

        def compute(p, car):
            lab_ref, pred_ref = labs[p], preds[p]

            def row_body(r, c5):
                s_acc = c5[0]
                g = list(c5[1:])
                for l in range(32):
                    lab16 = lab_ref[r, pl.ds(l * 16, 16)]
                    jrel = plsc.bitcast(lab16 - c_lo, jnp.uint32)
                    jsel = plsc.bitcast(
                        jnp.minimum(jrel, jnp.uint32(nslab)), jnp.int32)
                    row16 = (jsel << 3) + r
                    col16 = iota + (l * 16)
                    v16 = plsc.load_gather(pred_ref, [row16, col16])
                    g[l % 4] = g[l % 4] + v16
                    s_acc = s_acc + lab16
                return (s_acc, g[0], g[1], g[2], g[3])

            return lax.fori_loop(0, WIN_ROWS, row_body, car)

        def zero_junk(p):
            zf = jnp.zeros((16,), jnp.float32)

            def zrow(r, _):
                for l in range(32):
                    preds[p][nslab * WIN_ROWS + r, pl.ds(l * 16, 16)] = zf
                return 0

            lax.fori_loop(0, WIN_ROWS, zrow, 0)

        for p in range(NBUF):
            zero_junk(p)
            start(p, p)

        zf = jnp.zeros((16,), jnp.float32)
        car = (jnp.zeros((16,), jnp.int32), zf, zf, zf, zf)

        def outer(w4, car):
            for p in range(NBUF):
                w = w4 * NBUF + p
                wait(p)
                car = compute(p, car)

                @pl.when(w + NBUF < NWIN)
                def _():
                    start(w + NBUF, p)
            return car

        car = lax.fori_loop(0, NWIN // NBUF, outer, car)

        s_acc, g0, g1, g2, g3 = car
        g_tot = (g0 + g1) + (g2 + g3)
        cg_v = jnp.full((16,), cg, jnp.int32)
        s_fin = jnp.where(cg_v == 0, s_acc, 0).astype(jnp.float32)
        stg_g[...] = g_tot
        stg_s[...] = s_fin
        pltpu.sync_copy(stg_g, outg_ref.at[wid])
        pltpu.sync_copy(stg_s, outs_ref.at[wid])

    return body


def _tc_dense_kernel(yp_ref, yt_ref, g_ref, acc_ref):
    b = pl.program_id(0)

    @pl.when(b == 0)
    def _():
        acc_ref[...] = jnp.zeros_like(acc_ref)

    yt = yt_ref[0, 0]
    acc = acc_ref[...]
    for c in range(CT):
        yp = yp_ref[0, c]
        acc = acc + jnp.where(yt == c, yp, 0.0)
    acc_ref[...] = acc

    @pl.when(b == pl.num_programs(0) - 1)
    def _():
        g_ref[0, 0] = jnp.sum(acc_ref[...])


def _combine_kernel(g_ref, s_ref, gtc_ref, o_ref):
    g = jnp.sum(g_ref[...]) + gtc_ref[0, 0]
    s = jnp.sum(s_ref[...])
    denom = 10.0 * s + BETA * (float(P) - g) + EPS
    o_ref[0, 0] = 1.0 - s / denom


def kernel(y_pred, y_true):
    info = plsc.get_sparse_core_info()
    nw = info.num_cores * info.num_subcores

    yp2 = y_pred.reshape(B * C * 512, 512)
    yt2 = y_true.reshape(B * 512, 512)

    gpart, spart = _sc_partials(nw)(yp2, yt2)

    gtc = pl.pallas_call(
        _tc_dense_kernel,
        grid=(B,),
        in_specs=[
            pl.BlockSpec((1, CT, 512, 512), lambda b: (b, 0, 0, 0)),
            pl.BlockSpec((1, 1, 512, 512), lambda b: (b, 0, 0, 0)),
        ],
        out_specs=pl.BlockSpec((1, 1), lambda b: (0, 0),
                               memory_space=pltpu.SMEM),
        out_shape=jax.ShapeDtypeStruct((1, 1), jnp.float32),
        scratch_shapes=[pltpu.VMEM((512, 512), jnp.float32)],
    )(y_pred, y_true)

    out = pl.pallas_call(
        _combine_kernel,
        in_specs=[
            pl.BlockSpec((nw, 16), lambda: (0, 0)),
            pl.BlockSpec((nw, 16), lambda: (0, 0)),
            pl.BlockSpec(memory_space=pltpu.SMEM),
        ],
        out_shape=jax.ShapeDtypeStruct((1, 1), jnp.float32),
        out_specs=pl.BlockSpec(memory_space=pltpu.SMEM),
    )(gpart, spart, gtc)
    return out.reshape(())


# speedup vs baseline: 1.7750x; 1.0930x over previous
import functools

import jax
import jax.numpy as jnp
from jax import lax
from jax.experimental import pallas as pl
from jax.experimental.pallas import tpu as pltpu
from jax.experimental.pallas import tpu_sc as plsc

ALPHA = 0.5
BETA = 0.5
EPS = 1e-06
C = 19
B = 8
HW = 512 * 512
P = B * HW

WIN_ROWS = 8
WIN = WIN_ROWS * 512
NWIN = HW // WIN
NBUF = 4
CT = 15


def _sc_partials(nw):
    nq = nw // B
    qrows = 512 // nq
    nwin_w = qrows // WIN_ROWS
    nslab = C - CT

    mesh = plsc.VectorSubcoreMesh(core_axis_name="c", subcore_axis_name="s")

    @functools.partial(
        pl.kernel,
        out_type=[
            jax.ShapeDtypeStruct((nw, 16), jnp.float32),
            jax.ShapeDtypeStruct((nw, 16), jnp.float32),
        ],
        mesh=mesh,
        compiler_params=pltpu.CompilerParams(needs_layout_passes=False),
        scratch_types=(
            [pltpu.VMEM((WIN_ROWS, 512), jnp.int32) for _ in range(NBUF)]
            + [pltpu.VMEM(((nslab + 1) * WIN_ROWS, 512), jnp.float32)
               for _ in range(NBUF)]
            + [pltpu.VMEM((16,), jnp.float32), pltpu.VMEM((16,), jnp.float32)]
            + [pltpu.SemaphoreType.DMA for _ in range(NBUF)]
        ),
    )
    def body(yp_ref, yt_ref, outg_ref, outs_ref, *refs):
        labs = refs[:NBUF]
        preds = refs[NBUF:2 * NBUF]
        stg_g, stg_s = refs[2 * NBUF], refs[2 * NBUF + 1]
        sems = refs[2 * NBUF + 2:]

        nc = jax.lax.axis_size("c")
        wid = lax.axis_index("s") * nc + lax.axis_index("c")
        b = wid // nq
        q = wid % nq
        lab_row0 = b * 512 + q * qrows

        iota = lax.iota(jnp.int32, 16)

        def start(w, p):
            rb = w * WIN_ROWS
            pltpu.async_copy(
                yt_ref.at[pl.ds(lab_row0 + rb, WIN_ROWS)], labs[p], sems[p])
            for j in range(nslab):
                row = (b * C + (CT + j)) * 512 + q * qrows + rb
                pltpu.async_copy(
                    yp_ref.at[pl.ds(row, WIN_ROWS)],
                    preds[p].at[pl.ds(j * WIN_ROWS, WIN_ROWS)], sems[p])

        def wait(p):
            pltpu.make_async_copy(
                yp_ref.at[pl.ds(0, nslab * WIN_ROWS)],
                preds[p].at[pl.ds(0, nslab * WIN_ROWS)], sems[p]).wait()
            pltpu.make_async_copy(
                yt_ref.at[pl.ds(0, WIN_ROWS)], labs[p], sems[p]).wait()

        def compute(p, car):
            lab_ref, pred_ref = labs[p], preds[p]

            def row_body(r, c5):
                s_acc = c5[0]
                g = list(c5[1:])
                for l in range(32):
                    lab16 = lab_ref[r, pl.ds(l * 16, 16)]
                    jrel = plsc.bitcast(lab16 - CT, jnp.uint32)
                    jsel = plsc.bitcast(
                        jnp.minimum(jrel, jnp.uint32(nslab)), jnp.int32)
                    row16 = (jsel << 3) + r
                    col16 = iota + (l * 16)
                    v16 = plsc.load_gather(pred_ref, [row16, col16])
                    g[l % 4] = g[l % 4] + v16
                    s_acc = s_acc + lab16
                return (s_acc, g[0], g[1], g[2], g[3])

            return lax.fori_loop(0, WIN_ROWS, row_body, car)

        def zero_junk(p):
            zf = jnp.zeros((16,), jnp.float32)

            def zrow(r, _):
                for l in range(32):
                    preds[p][nslab * WIN_ROWS + r, pl.ds(l * 16, 16)] = zf
                return 0

            lax.fori_loop(0, WIN_ROWS, zrow, 0)

        for p in range(NBUF):
            zero_junk(p)
            start(p, p)

        zf = jnp.zeros((16,), jnp.float32)
        car = (jnp.zeros((16,), jnp.int32), zf, zf, zf, zf)

        def outer(w4, car):
            for p in range(NBUF):
                w = w4 * NBUF + p
                wait(p)
                car = compute(p, car)

                @pl.when(w + NBUF < nwin_w)
                def _():
                    start(w + NBUF, p)
            return car

        car = lax.fori_loop(0, nwin_w // NBUF, outer, car)

        s_acc, g0, g1, g2, g3 = car
        g_tot = (g0 + g1) + (g2 + g3)
        stg_g[...] = g_tot
        stg_s[...] = s_acc.astype(jnp.float32)
        pltpu.sync_copy(stg_g, outg_ref.at[wid])
        pltpu.sync_copy(stg_s, outs_ref.at[wid])

    return body


def _tc_dense_kernel(yp_ref, yt_ref, g_ref, acc_ref):
    b = pl.program_id(0)

    @pl.when(b == 0)
    def _():
        acc_ref[...] = jnp.zeros_like(acc_ref)

    yt = yt_ref[0, 0]
    acc = acc_ref[...]
    for c in range(CT):
        yp = yp_ref[0, c]
        acc = acc + jnp.where(yt == c, yp, 0.0)
    acc_ref[...] = acc

    @pl.when(b == pl.num_programs(0) - 1)
    def _():
        g_ref[0, 0] = jnp.sum(acc_ref[...])


def _combine_kernel(g_ref, s_ref, gtc_ref, o_ref):
    g = jnp.sum(g_ref[...]) + gtc_ref[0, 0]
    s = jnp.sum(s_ref[...])
    denom = 10.0 * s + BETA * (float(P) - g) + EPS
    o_ref[0, 0] = 1.0 - s / denom


def kernel(y_pred, y_true):
    info = plsc.get_sparse_core_info()
    nw = info.num_cores * info.num_subcores

    yp2 = y_pred.reshape(B * C * 512, 512)
    yt2 = y_true.reshape(B * 512, 512)

    gpart, spart = _sc_partials(nw)(yp2, yt2)

    gtc = pl.pallas_call(
        _tc_dense_kernel,
        grid=(B,),
        in_specs=[
            pl.BlockSpec((1, CT, 512, 512), lambda b: (b, 0, 0, 0)),
            pl.BlockSpec((1, 1, 512, 512), lambda b: (b, 0, 0, 0)),
        ],
        out_specs=pl.BlockSpec((1, 1), lambda b: (0, 0),
                               memory_space=pltpu.SMEM),
        out_shape=jax.ShapeDtypeStruct((1, 1), jnp.float32),
        scratch_shapes=[pltpu.VMEM((512, 512), jnp.float32)],
    )(y_pred, y_true)

    out = pl.pallas_call(
        _combine_kernel,
        in_specs=[
            pl.BlockSpec((nw, 16), lambda: (0, 0)),
            pl.BlockSpec((nw, 16), lambda: (0, 0)),
            pl.BlockSpec(memory_space=pltpu.SMEM),
        ],
        out_shape=jax.ShapeDtypeStruct((1, 1), jnp.float32),
        out_specs=pl.BlockSpec(memory_space=pltpu.SMEM),
    )(gpart, spart, gtc)
    return out.reshape(())
